# baseline (speedup 1.0000x reference)
import dataclasses
import functools

import jax
import jax.numpy as jnp
from jax import lax
from jax.experimental import pallas as pl
from jax.experimental.pallas import tpu as pltpu
from jax.experimental.pallas import tpu_sc as plsc

H = 128
MAX_NB = 32
NC = 2
NS = 16
NW = NC * NS
LANES = 16


def _mm_t(x, w):
    return lax.dot_general(x, w, dimension_numbers=(((1,), (1,)), ((), ())),
                           preferred_element_type=jnp.float32)


def _gather_sum_sc(table, idx_flat, atoms_pad):
    w2 = H // 2
    n_rows = table.shape[0]
    rpt = n_rows // NS
    sr = 25
    assert n_rows % NS == 0 and rpt % sr == 0
    apw = atoms_pad // NW
    ipw = apw * MAX_NB
    ca = 8
    ci = ca * MAX_NB
    gi = 128
    nchunk = apw // ca
    assert atoms_pad % NW == 0 and apw % ca == 0 and nchunk % 2 == 0

    mesh = plsc.VectorSubcoreMesh(core_axis_name="c", subcore_axis_name="s",
                                  num_cores=NC, num_subcores=NS)

    nbuf = 2
    nstage = rpt // sr
    assert nchunk % nbuf == 0

    def body(table_hbm, idx_hbm, out_hbm, idx_v, shared_tab,
             sf0, sf1, sp0, sp1, isem0, isem1, psem0, psem1, *bufs_sems):
        bufs = bufs_sems[:nbuf]
        ostgs = bufs_sems[nbuf:2 * nbuf]
        sems = bufs_sems[2 * nbuf:3 * nbuf]
        osems = bufs_sems[3 * nbuf:]
        sfs, sps = (sf0, sf1), (sp0, sp1)
        isems, psems = (isem0, isem1), (psem0, psem1)
        sid = lax.axis_index("s")
        wid = sid * NC + lax.axis_index("c")
        pltpu.sync_copy(idx_hbm.at[pl.ds(wid * ipw, ipw)], idx_v)

        def srows(g):
            return pl.ds(sid * rpt + g * sr, sr)

        pltpu.async_copy(table_hbm.at[srows(0)], sfs[0], isems[0])
        for g in range(nstage):
            p = g % 2
            pltpu.make_async_copy(
                table_hbm.at[srows(g)], sfs[p], isems[p]).wait()
            if g + 1 < nstage:
                pltpu.async_copy(table_hbm.at[srows(g + 1)], sfs[1 - p],
                                 isems[1 - p])
            if g >= 2:
                pltpu.make_async_copy(
                    sps[p], shared_tab.at[srows(g)], psems[p]).wait()

            @pl.loop(0, sr, step=5)
            def _(r0):
                for dr in range(5):
                    r = r0 + dr
                    for h in range(w2 // LANES):
                        a = sfs[p][r, pl.ds(2 * h * LANES, LANES)]
                        b = sfs[p][r, pl.ds((2 * h + 1) * LANES, LANES)]
                        packed = plsc.pack(a, b,
                                           format=plsc.PackFormat.INTERLEAVED)
                        sps[p][r, pl.ds(h * LANES, LANES)] = plsc.bitcast(
                            packed, jnp.int32)

            pltpu.async_copy(sps[p], shared_tab.at[srows(g)], psems[p])
        for p in range(2):
            pltpu.make_async_copy(
                sps[p], shared_tab.at[srows(0)], psems[p]).wait()
        plsc.subcore_barrier()

        def start(c, buf, sem):
            for k in range(ci // gi):
                pltpu.async_copy(
                    shared_tab.at[idx_v.at[pl.ds(c * ci + k * gi, gi)]],
                    buf.at[pl.ds(k * gi, gi)], sem)

        def wait(buf, sem):
            pltpu.make_async_copy(
                shared_tab.at[idx_v.at[pl.ds(0, ci)]], buf, sem).wait()

        nh = w2 // LANES

        def load_bf(buf, row, h):
            return plsc.bitcast(buf[row, pl.ds(h * LANES, LANES)], jnp.bfloat16)

        def accum(buf, ostg):
            @pl.loop(0, ca)
            def _(a):
                base = a * MAX_NB

                def rbody(r, accs):
                    return tuple(accs[h] + load_bf(buf, base + r, h)
                                 for h in range(nh))

                accs = tuple(load_bf(buf, base, h) for h in range(nh))
                accs = lax.fori_loop(1, MAX_NB, rbody, accs, unroll=4)
                for h in range(nh):
                    av, bv = plsc.unpack(accs[h],
                                         format=plsc.PackFormat.INTERLEAVED)
                    ostg[a, pl.ds(2 * h * LANES, LANES)] = av
                    ostg[a, pl.ds((2 * h + 1) * LANES, LANES)] = bv

        def owait(ostg, osem):
            pltpu.make_async_copy(ostg, out_hbm.at[pl.ds(0, ca)], osem).wait()

        for b in range(nbuf):
            start(b, bufs[b], sems[b])

        @pl.loop(0, nchunk, step=nbuf)
        def _(c):
            for b in range(nbuf):
                wait(bufs[b], sems[b])

                @pl.when(c + b >= nbuf)
                def _(b=b):
                    owait(ostgs[b], osems[b])

                accum(bufs[b], ostgs[b])
                pltpu.async_copy(
                    ostgs[b],
                    out_hbm.at[pl.ds(wid * apw + (c + b) * ca, ca)], osems[b])

                @pl.when(c + b + nbuf < nchunk)
                def _(b=b):
                    start(c + b + nbuf, bufs[b], sems[b])

        for b in range(nbuf):
            owait(ostgs[b], osems[b])

    cp = pltpu.CompilerParams(use_tc_tiling_on_sc=False)
    if "needs_layout_passes" in pltpu.CompilerParams.__dataclass_fields__:
        cp = dataclasses.replace(cp, needs_layout_passes=False)
    kfn = pl.kernel(
        body,
        out_type=jax.ShapeDtypeStruct((atoms_pad, H), jnp.float32),
        compiler_params=cp,
        mesh=mesh,
        scratch_types=(
            [pltpu.VMEM((ipw,), jnp.int32),
             pltpu.VMEM_SHARED((n_rows, w2), jnp.int32),
             pltpu.VMEM((sr, H), jnp.float32),
             pltpu.VMEM((sr, H), jnp.float32),
             pltpu.VMEM((sr, w2), jnp.int32),
             pltpu.VMEM((sr, w2), jnp.int32),
             pltpu.SemaphoreType.DMA,
             pltpu.SemaphoreType.DMA,
             pltpu.SemaphoreType.DMA,
             pltpu.SemaphoreType.DMA]
            + [pltpu.VMEM((ci, w2), jnp.int32) for _ in range(nbuf)]
            + [pltpu.VMEM((ca, H), jnp.float32) for _ in range(nbuf)]
            + [pltpu.SemaphoreType.DMA for _ in range(2 * nbuf)]
        ),
    )
    return kfn(table, idx_flat)


def _tc_input(f_atoms, W_i):
    n = f_atoms.shape[0]

    def body(f_ref, w_ref, inp_ref, msg_ref):
        inp = _mm_t(f_ref[...], w_ref[...])
        inp_ref[...] = inp
        msg_ref[...] = jnp.maximum(inp, 0.0)

    return pl.pallas_call(
        body,
        out_shape=(jax.ShapeDtypeStruct((n, H), jnp.float32),
                   jax.ShapeDtypeStruct((n, H), jnp.float32)),
    )(f_atoms, W_i)


def _tc_update(inp, msum_pad, W_h):
    n = inp.shape[0]

    def body(inp_ref, ms_ref, w_ref, out_ref):
        msum = ms_ref[...][:n]
        out_ref[...] = jnp.maximum(
            inp_ref[...] + _mm_t(msum, w_ref[...]), 0.0)

    return pl.pallas_call(
        body,
        out_shape=jax.ShapeDtypeStruct((n, H), jnp.float32),
    )(inp, msum_pad, W_h)


def _tc_head(f_atoms, am, mol_ids2d, e0, e1, W_o, b_o,
             Wf_i, bf_i, Wf1, bf1, Wf2, bf2, n_mols):
    n = f_atoms.shape[0]
    npairs = e0.shape[0]

    def body(f_ref, am_ref, mol_ref, e0_ref, e1_ref, wo_ref, bo_ref,
             wfi_ref, bfi_ref, wf1_ref, bf1_ref, wf2_ref, bf2_ref, out_ref):
        am = am_ref[...][:n]
        wo = wo_ref[...]
        ah = jnp.maximum(
            _mm_t(f_ref[...], wo[:, :H]) + _mm_t(am, wo[:, H:])
            + bo_ref[...], 0.0)
        seg = lax.broadcasted_iota(jnp.int32, (n_mols, n), 0)
        ohm = (seg == mol_ref[...]).astype(jnp.float32)
        sums = jnp.dot(ohm, ah, preferred_element_type=jnp.float32)
        counts = jnp.sum(ohm, axis=1, keepdims=True)
        mv = sums / jnp.maximum(counts, 1.0)

        pid = lax.broadcasted_iota(jnp.int32, (npairs, n_mols), 1)
        oh0 = (pid == e0_ref[...]).astype(jnp.float32)
        oh1 = (pid == e1_ref[...]).astype(jnp.float32)
        v1 = jnp.dot(oh0, mv, preferred_element_type=jnp.float32)
        v2 = jnp.dot(oh1, mv, preferred_element_type=jnp.float32)

        wfi = wfi_ref[...]
        fused = (_mm_t(v1 + v2, wfi[:, :H]) + _mm_t(v1 * v2, wfi[:, H:2 * H])
                 + _mm_t(v1, wfi[:, 2 * H:3 * H]) + _mm_t(v2, wfi[:, 3 * H:])
                 + bfi_ref[...])
        x = jnp.maximum(fused, 0.0)
        x = jnp.maximum(_mm_t(x, wf1_ref[...]) + bf1_ref[...], 0.0)
        logit = jnp.sum(x * wf2_ref[...], axis=1, keepdims=True) + bf2_ref[0, 0]
        out_ref[...] = jax.nn.sigmoid(logit)

    return pl.pallas_call(
        body,
        out_shape=jax.ShapeDtypeStruct((npairs, 1), jnp.float32),
    )(f_atoms, am, mol_ids2d, e0, e1, W_o, b_o,
      Wf_i, bf_i, Wf1, bf1, Wf2, bf2)


def kernel(f_atoms, a_neighbors, mol_ids, batch_edges, W_i, W_h, W_o, b_o,
           Wf_i, bf_i, Wf1, bf1, Wf2, bf2):
    n = f_atoms.shape[0]
    n_mols = 256
    atoms_pad = -(-n // (NW * 8)) * (NW * 8)

    nb = a_neighbors.astype(jnp.int32)
    nb_pad = jnp.pad(nb, ((0, atoms_pad - n), (0, 0)))
    idx_flat = nb_pad.reshape(-1)

    inp, msgp = _tc_input(f_atoms, W_i)
    msump = _gather_sum_sc(msgp, idx_flat, atoms_pad)
    msg2p = _tc_update(inp, msump, W_h)
    am2 = _gather_sum_sc(msg2p, idx_flat, atoms_pad)

    mol_ids2d = mol_ids.astype(jnp.int32).reshape(1, n)
    e0 = batch_edges[0].astype(jnp.int32).reshape(-1, 1)
    e1 = batch_edges[1].astype(jnp.int32).reshape(-1, 1)

    preds = _tc_head(f_atoms, am2, mol_ids2d, e0, e1, W_o,
                     b_o.reshape(1, H), Wf_i, bf_i.reshape(1, -1),
                     Wf1, bf1.reshape(1, -1), Wf2, bf2.reshape(1, -1), n_mols)
    return preds.reshape(-1)


# speedup vs baseline: 1.1721x; 1.1721x over previous
import dataclasses
import functools

import jax
import jax.numpy as jnp
from jax import lax
from jax.experimental import pallas as pl
from jax.experimental.pallas import tpu as pltpu
from jax.experimental.pallas import tpu_sc as plsc

H = 128
MAX_NB = 32
NC = 2
NS = 16
NW = NC * NS
LANES = 16


def _mm_t(x, w):
    return lax.dot_general(x, w, dimension_numbers=(((1,), (1,)), ((), ())),
                           preferred_element_type=jnp.float32)


def _gather_sum_sc(table, idx_flat, atoms_pad):
    w2 = H // 2
    n_rows = table.shape[0]
    rpt = n_rows // NS
    sr = 125
    assert n_rows % NS == 0 and rpt % sr == 0
    apw = atoms_pad // NW
    ipw = apw * MAX_NB
    ca = 4
    ci = ca * MAX_NB
    gi = 128
    nchunk = apw // ca
    assert atoms_pad % NW == 0 and apw % ca == 0 and nchunk % 2 == 0

    mesh = plsc.VectorSubcoreMesh(core_axis_name="c", subcore_axis_name="s",
                                  num_cores=NC, num_subcores=NS)

    nbuf = 2
    nstage = rpt // sr
    assert nchunk % nbuf == 0

    def body(table_hbm, idx_hbm, out_hbm, idx_v, shared_tab,
             sf0, sf1, sp0, sp1, isem0, isem1, psem0, psem1, *bufs_sems):
        bufs = bufs_sems[:nbuf]
        ostgs = bufs_sems[nbuf:2 * nbuf]
        sems = bufs_sems[2 * nbuf:3 * nbuf]
        osems = bufs_sems[3 * nbuf:]
        sfs, sps = (sf0, sf1), (sp0, sp1)
        isems, psems = (isem0, isem1), (psem0, psem1)
        sid = lax.axis_index("s")
        wid = sid * NC + lax.axis_index("c")
        pltpu.sync_copy(idx_hbm.at[pl.ds(wid * ipw, ipw)], idx_v)

        def srows(g):
            return pl.ds(sid * rpt + g * sr, sr)

        pltpu.async_copy(table_hbm.at[srows(0)], sfs[0], isems[0])
        for g in range(nstage):
            p = g % 2
            pltpu.make_async_copy(
                table_hbm.at[srows(g)], sfs[p], isems[p]).wait()
            if g + 1 < nstage:
                pltpu.async_copy(table_hbm.at[srows(g + 1)], sfs[1 - p],
                                 isems[1 - p])
            if g >= 2:
                pltpu.make_async_copy(
                    sps[p], shared_tab.at[srows(g)], psems[p]).wait()

            @pl.loop(0, sr, step=5)
            def _(r0):
                for dr in range(5):
                    r = r0 + dr
                    for h in range(w2 // LANES):
                        a = sfs[p][r, pl.ds(2 * h * LANES, LANES)]
                        b = sfs[p][r, pl.ds((2 * h + 1) * LANES, LANES)]
                        packed = plsc.pack(a, b,
                                           format=plsc.PackFormat.INTERLEAVED)
                        sps[p][r, pl.ds(h * LANES, LANES)] = plsc.bitcast(
                            packed, jnp.int32)

            pltpu.async_copy(sps[p], shared_tab.at[srows(g)], psems[p])
        for p in range(2):
            pltpu.make_async_copy(
                sps[p], shared_tab.at[srows(0)], psems[p]).wait()
        plsc.subcore_barrier()

        def start(c, buf, sem):
            for k in range(ci // gi):
                pltpu.async_copy(
                    shared_tab.at[idx_v.at[pl.ds(c * ci + k * gi, gi)]],
                    buf.at[pl.ds(k * gi, gi)], sem)

        def wait(buf, sem):
            pltpu.make_async_copy(
                shared_tab.at[idx_v.at[pl.ds(0, ci)]], buf, sem).wait()

        nh = w2 // LANES

        def load_bf(buf, row, h):
            return plsc.bitcast(buf[row, pl.ds(h * LANES, LANES)], jnp.bfloat16)

        def accum(buf, ostg):
            @pl.loop(0, ca)
            def _(a):
                base = a * MAX_NB

                def rbody(r, accs):
                    return tuple(accs[h] + load_bf(buf, base + r, h)
                                 for h in range(nh))

                accs = tuple(load_bf(buf, base, h) for h in range(nh))
                accs = lax.fori_loop(1, MAX_NB, rbody, accs, unroll=4)
                for h in range(nh):
                    av, bv = plsc.unpack(accs[h],
                                         format=plsc.PackFormat.INTERLEAVED)
                    ostg[a, pl.ds(2 * h * LANES, LANES)] = av
                    ostg[a, pl.ds((2 * h + 1) * LANES, LANES)] = bv

        def owait(ostg, osem):
            pltpu.make_async_copy(ostg, out_hbm.at[pl.ds(0, ca)], osem).wait()

        for b in range(nbuf):
            start(b, bufs[b], sems[b])

        @pl.loop(0, nchunk, step=nbuf)
        def _(c):
            for b in range(nbuf):
                wait(bufs[b], sems[b])

                @pl.when(c + b >= nbuf)
                def _(b=b):
                    owait(ostgs[b], osems[b])

                accum(bufs[b], ostgs[b])
                pltpu.async_copy(
                    ostgs[b],
                    out_hbm.at[pl.ds(wid * apw + (c + b) * ca, ca)], osems[b])

                @pl.when(c + b + nbuf < nchunk)
                def _(b=b):
                    start(c + b + nbuf, bufs[b], sems[b])

        for b in range(nbuf):
            owait(ostgs[b], osems[b])

    cp = pltpu.CompilerParams(use_tc_tiling_on_sc=False)
    if "needs_layout_passes" in pltpu.CompilerParams.__dataclass_fields__:
        cp = dataclasses.replace(cp, needs_layout_passes=False)
    kfn = pl.kernel(
        body,
        out_type=jax.ShapeDtypeStruct((atoms_pad, H), jnp.float32),
        compiler_params=cp,
        mesh=mesh,
        scratch_types=(
            [pltpu.VMEM((ipw,), jnp.int32),
             pltpu.VMEM_SHARED((n_rows, w2), jnp.int32),
             pltpu.VMEM((sr, H), jnp.float32),
             pltpu.VMEM((sr, H), jnp.float32),
             pltpu.VMEM((sr, w2), jnp.int32),
             pltpu.VMEM((sr, w2), jnp.int32),
             pltpu.SemaphoreType.DMA,
             pltpu.SemaphoreType.DMA,
             pltpu.SemaphoreType.DMA,
             pltpu.SemaphoreType.DMA]
            + [pltpu.VMEM((ci, w2), jnp.int32) for _ in range(nbuf)]
            + [pltpu.VMEM((ca, H), jnp.float32) for _ in range(nbuf)]
            + [pltpu.SemaphoreType.DMA for _ in range(2 * nbuf)]
        ),
    )
    return kfn(table, idx_flat)


def _tc_input(f_atoms, W_i):
    n = f_atoms.shape[0]

    def body(f_ref, w_ref, inp_ref, msg_ref):
        inp = _mm_t(f_ref[...], w_ref[...])
        inp_ref[...] = inp
        msg_ref[...] = jnp.maximum(inp, 0.0)

    return pl.pallas_call(
        body,
        out_shape=(jax.ShapeDtypeStruct((n, H), jnp.float32),
                   jax.ShapeDtypeStruct((n, H), jnp.float32)),
    )(f_atoms, W_i)


def _tc_update(inp, msum_pad, W_h):
    n = inp.shape[0]

    def body(inp_ref, ms_ref, w_ref, out_ref):
        msum = ms_ref[...][:n]
        out_ref[...] = jnp.maximum(
            inp_ref[...] + _mm_t(msum, w_ref[...]), 0.0)

    return pl.pallas_call(
        body,
        out_shape=jax.ShapeDtypeStruct((n, H), jnp.float32),
    )(inp, msum_pad, W_h)


def _tc_head(f_atoms, am, mol_ids2d, e0, e1, W_o, b_o,
             Wf_i, bf_i, Wf1, bf1, Wf2, bf2, n_mols):
    n = f_atoms.shape[0]
    npairs = e0.shape[0]

    def body(f_ref, am_ref, mol_ref, e0_ref, e1_ref, wo_ref, bo_ref,
             wfi_ref, bfi_ref, wf1_ref, bf1_ref, wf2_ref, bf2_ref, out_ref):
        am = am_ref[...][:n]
        wo = wo_ref[...]
        ah = jnp.maximum(
            _mm_t(f_ref[...], wo[:, :H]) + _mm_t(am, wo[:, H:])
            + bo_ref[...], 0.0)
        seg = lax.broadcasted_iota(jnp.int32, (n_mols, n), 0)
        ohm = (seg == mol_ref[...]).astype(jnp.float32)
        sums = jnp.dot(ohm, ah, preferred_element_type=jnp.float32)
        counts = jnp.sum(ohm, axis=1, keepdims=True)
        mv = sums / jnp.maximum(counts, 1.0)

        pid = lax.broadcasted_iota(jnp.int32, (npairs, n_mols), 1)
        oh0 = (pid == e0_ref[...]).astype(jnp.float32)
        oh1 = (pid == e1_ref[...]).astype(jnp.float32)
        v1 = jnp.dot(oh0, mv, preferred_element_type=jnp.float32)
        v2 = jnp.dot(oh1, mv, preferred_element_type=jnp.float32)

        wfi = wfi_ref[...]
        fused = (_mm_t(v1 + v2, wfi[:, :H]) + _mm_t(v1 * v2, wfi[:, H:2 * H])
                 + _mm_t(v1, wfi[:, 2 * H:3 * H]) + _mm_t(v2, wfi[:, 3 * H:])
                 + bfi_ref[...])
        x = jnp.maximum(fused, 0.0)
        x = jnp.maximum(_mm_t(x, wf1_ref[...]) + bf1_ref[...], 0.0)
        logit = jnp.sum(x * wf2_ref[...], axis=1, keepdims=True) + bf2_ref[0, 0]
        out_ref[...] = jax.nn.sigmoid(logit)

    return pl.pallas_call(
        body,
        out_shape=jax.ShapeDtypeStruct((npairs, 1), jnp.float32),
    )(f_atoms, am, mol_ids2d, e0, e1, W_o, b_o,
      Wf_i, bf_i, Wf1, bf1, Wf2, bf2)


def kernel(f_atoms, a_neighbors, mol_ids, batch_edges, W_i, W_h, W_o, b_o,
           Wf_i, bf_i, Wf1, bf1, Wf2, bf2):
    n = f_atoms.shape[0]
    n_mols = 256
    atoms_pad = -(-n // (NW * 8)) * (NW * 8)

    nb = a_neighbors.astype(jnp.int32)
    nb_pad = jnp.pad(nb, ((0, atoms_pad - n), (0, 0)))
    idx_flat = nb_pad.reshape(-1)

    inp, msgp = _tc_input(f_atoms, W_i)
    msump = _gather_sum_sc(msgp, idx_flat, atoms_pad)
    msg2p = _tc_update(inp, msump, W_h)
    am2 = _gather_sum_sc(msg2p, idx_flat, atoms_pad)

    mol_ids2d = mol_ids.astype(jnp.int32).reshape(1, n)
    e0 = batch_edges[0].astype(jnp.int32).reshape(-1, 1)
    e1 = batch_edges[1].astype(jnp.int32).reshape(-1, 1)

    preds = _tc_head(f_atoms, am2, mol_ids2d, e0, e1, W_o,
                     b_o.reshape(1, H), Wf_i, bf_i.reshape(1, -1),
                     Wf1, bf1.reshape(1, -1), Wf2, bf2.reshape(1, -1), n_mols)
    return preds.reshape(-1)


# speedup vs baseline: 1.1831x; 1.0094x over previous
import dataclasses
import functools

import jax
import jax.numpy as jnp
from jax import lax
from jax.experimental import pallas as pl
from jax.experimental.pallas import tpu as pltpu
from jax.experimental.pallas import tpu_sc as plsc

H = 128
MAX_NB = 32
NC = 2
NS = 16
NW = NC * NS
LANES = 16


def _mm_t(x, w):
    return lax.dot_general(x, w, dimension_numbers=(((1,), (1,)), ((), ())),
                           preferred_element_type=jnp.float32)


def _gather_sum_sc(table, idx_flat, atoms_pad):
    w2 = H // 2
    n_rows = table.shape[0]
    rpt = n_rows // NS
    sr = 125
    assert n_rows % NS == 0 and rpt % sr == 0
    apw = atoms_pad // NW
    ipw = apw * MAX_NB
    ca = 4
    ci = ca * MAX_NB
    gi = 128
    nchunk = apw // ca
    assert atoms_pad % NW == 0 and apw % ca == 0 and nchunk % 2 == 0

    mesh = plsc.VectorSubcoreMesh(core_axis_name="c", subcore_axis_name="s",
                                  num_cores=NC, num_subcores=NS)

    nbuf = 2
    nstage = rpt // sr
    assert nchunk % nbuf == 0

    def body(table_hbm, idx_hbm, out_hbm, idx_v, shared_tab,
             sf0, sf1, sp0, sp1, isem0, isem1, psem0, psem1, xsem, *bufs_sems):
        bufs = bufs_sems[:nbuf]
        ostgs = bufs_sems[nbuf:2 * nbuf]
        sems = bufs_sems[2 * nbuf:3 * nbuf]
        osems = bufs_sems[3 * nbuf:]
        sfs, sps = (sf0, sf1), (sp0, sp1)
        isems, psems = (isem0, isem1), (psem0, psem1)
        sid = lax.axis_index("s")
        wid = sid * NC + lax.axis_index("c")
        pltpu.async_copy(idx_hbm.at[pl.ds(wid * ipw, ipw)], idx_v, xsem)

        def srows(g):
            return pl.ds(sid * rpt + g * sr, sr)

        pltpu.async_copy(table_hbm.at[srows(0)], sfs[0], isems[0])
        for g in range(nstage):
            p = g % 2
            pltpu.make_async_copy(
                table_hbm.at[srows(g)], sfs[p], isems[p]).wait()
            if g + 1 < nstage:
                pltpu.async_copy(table_hbm.at[srows(g + 1)], sfs[1 - p],
                                 isems[1 - p])
            if g >= 2:
                pltpu.make_async_copy(
                    sps[p], shared_tab.at[srows(g)], psems[p]).wait()

            @pl.loop(0, sr, step=5)
            def _(r0):
                for dr in range(5):
                    r = r0 + dr
                    for h in range(w2 // LANES):
                        a = sfs[p][r, pl.ds(2 * h * LANES, LANES)]
                        b = sfs[p][r, pl.ds((2 * h + 1) * LANES, LANES)]
                        packed = plsc.pack(a, b,
                                           format=plsc.PackFormat.INTERLEAVED)
                        sps[p][r, pl.ds(h * LANES, LANES)] = plsc.bitcast(
                            packed, jnp.int32)

            pltpu.async_copy(sps[p], shared_tab.at[srows(g)], psems[p])
        for p in range(2):
            pltpu.make_async_copy(
                sps[p], shared_tab.at[srows(0)], psems[p]).wait()
        pltpu.make_async_copy(
            idx_hbm.at[pl.ds(wid * ipw, ipw)], idx_v, xsem).wait()
        plsc.subcore_barrier()

        def start(c, buf, sem):
            for k in range(ci // gi):
                pltpu.async_copy(
                    shared_tab.at[idx_v.at[pl.ds(c * ci + k * gi, gi)]],
                    buf.at[pl.ds(k * gi, gi)], sem)

        def wait(buf, sem):
            pltpu.make_async_copy(
                shared_tab.at[idx_v.at[pl.ds(0, ci)]], buf, sem).wait()

        nh = w2 // LANES

        def load_bf(buf, row, h):
            return plsc.bitcast(buf[row, pl.ds(h * LANES, LANES)], jnp.bfloat16)

        def accum(buf, ostg):
            @pl.loop(0, ca)
            def _(a):
                base = a * MAX_NB

                def rbody(r, accs):
                    return tuple(accs[h] + load_bf(buf, base + r, h)
                                 for h in range(nh))

                accs = tuple(load_bf(buf, base, h) for h in range(nh))
                accs = lax.fori_loop(1, MAX_NB, rbody, accs, unroll=4)
                for h in range(nh):
                    av, bv = plsc.unpack(accs[h],
                                         format=plsc.PackFormat.INTERLEAVED)
                    ostg[a, pl.ds(2 * h * LANES, LANES)] = av
                    ostg[a, pl.ds((2 * h + 1) * LANES, LANES)] = bv

        def owait(ostg, osem):
            pltpu.make_async_copy(ostg, out_hbm.at[pl.ds(0, ca)], osem).wait()

        for b in range(nbuf):
            start(b, bufs[b], sems[b])

        @pl.loop(0, nchunk, step=nbuf)
        def _(c):
            for b in range(nbuf):
                wait(bufs[b], sems[b])

                @pl.when(c + b >= nbuf)
                def _(b=b):
                    owait(ostgs[b], osems[b])

                accum(bufs[b], ostgs[b])
                pltpu.async_copy(
                    ostgs[b],
                    out_hbm.at[pl.ds(wid * apw + (c + b) * ca, ca)], osems[b])

                @pl.when(c + b + nbuf < nchunk)
                def _(b=b):
                    start(c + b + nbuf, bufs[b], sems[b])

        for b in range(nbuf):
            owait(ostgs[b], osems[b])

    cp = pltpu.CompilerParams(use_tc_tiling_on_sc=False)
    if "needs_layout_passes" in pltpu.CompilerParams.__dataclass_fields__:
        cp = dataclasses.replace(cp, needs_layout_passes=False)
    kfn = pl.kernel(
        body,
        out_type=jax.ShapeDtypeStruct((atoms_pad, H), jnp.float32),
        compiler_params=cp,
        mesh=mesh,
        scratch_types=(
            [pltpu.VMEM((ipw,), jnp.int32),
             pltpu.VMEM_SHARED((n_rows, w2), jnp.int32),
             pltpu.VMEM((sr, H), jnp.float32),
             pltpu.VMEM((sr, H), jnp.float32),
             pltpu.VMEM((sr, w2), jnp.int32),
             pltpu.VMEM((sr, w2), jnp.int32),
             pltpu.SemaphoreType.DMA,
             pltpu.SemaphoreType.DMA,
             pltpu.SemaphoreType.DMA,
             pltpu.SemaphoreType.DMA,
             pltpu.SemaphoreType.DMA]
            + [pltpu.VMEM((ci, w2), jnp.int32) for _ in range(nbuf)]
            + [pltpu.VMEM((ca, H), jnp.float32) for _ in range(nbuf)]
            + [pltpu.SemaphoreType.DMA for _ in range(2 * nbuf)]
        ),
    )
    return kfn(table, idx_flat)


def _tc_input(f_atoms, W_i):
    n = f_atoms.shape[0]

    def body(f_ref, w_ref, inp_ref, msg_ref):
        inp = _mm_t(f_ref[...], w_ref[...])
        inp_ref[...] = inp
        msg_ref[...] = jnp.maximum(inp, 0.0)

    return pl.pallas_call(
        body,
        out_shape=(jax.ShapeDtypeStruct((n, H), jnp.float32),
                   jax.ShapeDtypeStruct((n, H), jnp.float32)),
    )(f_atoms, W_i)


def _tc_update(inp, msum_pad, W_h):
    n = inp.shape[0]

    def body(inp_ref, ms_ref, w_ref, out_ref):
        msum = ms_ref[...][:n]
        out_ref[...] = jnp.maximum(
            inp_ref[...] + _mm_t(msum, w_ref[...]), 0.0)

    return pl.pallas_call(
        body,
        out_shape=jax.ShapeDtypeStruct((n, H), jnp.float32),
    )(inp, msum_pad, W_h)


def _tc_head(f_atoms, am, mol_ids2d, e0, e1, W_o, b_o,
             Wf_i, bf_i, Wf1, bf1, Wf2, bf2, n_mols):
    n = f_atoms.shape[0]
    npairs = e0.shape[0]

    def body(f_ref, am_ref, mol_ref, e0_ref, e1_ref, wo_ref, bo_ref,
             wfi_ref, bfi_ref, wf1_ref, bf1_ref, wf2_ref, bf2_ref, out_ref):
        am = am_ref[...][:n]
        wo = wo_ref[...]
        ah = jnp.maximum(
            _mm_t(f_ref[...], wo[:, :H]) + _mm_t(am, wo[:, H:])
            + bo_ref[...], 0.0)
        seg = lax.broadcasted_iota(jnp.int32, (n_mols, n), 0)
        ohm = (seg == mol_ref[...]).astype(jnp.float32)
        sums = jnp.dot(ohm, ah, preferred_element_type=jnp.float32)
        counts = jnp.sum(ohm, axis=1, keepdims=True)
        mv = sums / jnp.maximum(counts, 1.0)

        pid = lax.broadcasted_iota(jnp.int32, (npairs, n_mols), 1)
        oh0 = (pid == e0_ref[...]).astype(jnp.float32)
        oh1 = (pid == e1_ref[...]).astype(jnp.float32)
        v1 = jnp.dot(oh0, mv, preferred_element_type=jnp.float32)
        v2 = jnp.dot(oh1, mv, preferred_element_type=jnp.float32)

        wfi = wfi_ref[...]
        fused = (_mm_t(v1 + v2, wfi[:, :H]) + _mm_t(v1 * v2, wfi[:, H:2 * H])
                 + _mm_t(v1, wfi[:, 2 * H:3 * H]) + _mm_t(v2, wfi[:, 3 * H:])
                 + bfi_ref[...])
        x = jnp.maximum(fused, 0.0)
        x = jnp.maximum(_mm_t(x, wf1_ref[...]) + bf1_ref[...], 0.0)
        logit = jnp.sum(x * wf2_ref[...], axis=1, keepdims=True) + bf2_ref[0, 0]
        out_ref[...] = jax.nn.sigmoid(logit)

    return pl.pallas_call(
        body,
        out_shape=jax.ShapeDtypeStruct((npairs, 1), jnp.float32),
    )(f_atoms, am, mol_ids2d, e0, e1, W_o, b_o,
      Wf_i, bf_i, Wf1, bf1, Wf2, bf2)


def kernel(f_atoms, a_neighbors, mol_ids, batch_edges, W_i, W_h, W_o, b_o,
           Wf_i, bf_i, Wf1, bf1, Wf2, bf2):
    n = f_atoms.shape[0]
    n_mols = 256
    atoms_pad = -(-n // (NW * 8)) * (NW * 8)

    nb = a_neighbors.astype(jnp.int32)
    nb_pad = jnp.pad(nb, ((0, atoms_pad - n), (0, 0)))
    idx_flat = nb_pad.reshape(-1)

    inp, msgp = _tc_input(f_atoms, W_i)
    msump = _gather_sum_sc(msgp, idx_flat, atoms_pad)
    msg2p = _tc_update(inp, msump, W_h)
    am2 = _gather_sum_sc(msg2p, idx_flat, atoms_pad)

    mol_ids2d = mol_ids.astype(jnp.int32).reshape(1, n)
    e0 = batch_edges[0].astype(jnp.int32).reshape(-1, 1)
    e1 = batch_edges[1].astype(jnp.int32).reshape(-1, 1)

    preds = _tc_head(f_atoms, am2, mol_ids2d, e0, e1, W_o,
                     b_o.reshape(1, H), Wf_i, bf_i.reshape(1, -1),
                     Wf1, bf1.reshape(1, -1), Wf2, bf2.reshape(1, -1), n_mols)
    return preds.reshape(-1)


# speedup vs baseline: 1.1846x; 1.0012x over previous
import dataclasses

import jax
import jax.numpy as jnp
from jax import lax
from jax.experimental import pallas as pl
from jax.experimental.pallas import tpu as pltpu
from jax.experimental.pallas import tpu_sc as plsc

H = 128
MAX_NB = 32
NC = 2
NS = 16
NW = NC * NS
LANES = 16


def _mm_t(x, w):
    return lax.dot_general(x, w, dimension_numbers=(((1,), (1,)), ((), ())),
                           preferred_element_type=jnp.float32)


def _gather_sum_sc(table, idx_flat, atoms_pad):
    w2 = H // 2
    n_rows = table.shape[0]
    rpt = n_rows // NS
    sr = 125
    assert n_rows % NS == 0 and rpt % sr == 0
    apw = atoms_pad // NW
    ipw = apw * MAX_NB
    ca = 4
    ci = ca * MAX_NB
    gi = 128
    nchunk = apw // ca
    assert atoms_pad % NW == 0 and apw % ca == 0 and nchunk % 2 == 0

    mesh = plsc.VectorSubcoreMesh(core_axis_name="c", subcore_axis_name="s",
                                  num_cores=NC, num_subcores=NS)

    nbuf = 2
    nstage = rpt // sr
    assert nchunk % nbuf == 0

    def body(table_hbm, idx_hbm, out_hbm, idx_v, shared_tab,
             sf0, sf1, sp0, sp1, isem0, isem1, psem0, psem1, xsem, *bufs_sems):
        bufs = bufs_sems[:nbuf]
        ostgs = bufs_sems[nbuf:2 * nbuf]
        sems = bufs_sems[2 * nbuf:3 * nbuf]
        osems = bufs_sems[3 * nbuf:]
        sfs, sps = (sf0, sf1), (sp0, sp1)
        isems, psems = (isem0, isem1), (psem0, psem1)
        sid = lax.axis_index("s")
        wid = sid * NC + lax.axis_index("c")
        pltpu.async_copy(idx_hbm.at[pl.ds(wid * ipw, ipw)], idx_v, xsem)

        def srows(g):
            return pl.ds(sid * rpt + g * sr, sr)

        pltpu.async_copy(table_hbm.at[srows(0)], sfs[0], isems[0])
        for g in range(nstage):
            p = g % 2
            pltpu.make_async_copy(
                table_hbm.at[srows(g)], sfs[p], isems[p]).wait()
            if g + 1 < nstage:
                pltpu.async_copy(table_hbm.at[srows(g + 1)], sfs[1 - p],
                                 isems[1 - p])
            if g >= 2:
                pltpu.make_async_copy(
                    sps[p], shared_tab.at[srows(g)], psems[p]).wait()

            @pl.loop(0, sr, step=5)
            def _(r0):
                for dr in range(5):
                    r = r0 + dr
                    for h in range(w2 // LANES):
                        a = sfs[p][r, pl.ds(2 * h * LANES, LANES)]
                        b = sfs[p][r, pl.ds((2 * h + 1) * LANES, LANES)]
                        packed = plsc.pack(a, b,
                                           format=plsc.PackFormat.INTERLEAVED)
                        sps[p][r, pl.ds(h * LANES, LANES)] = plsc.bitcast(
                            packed, jnp.int32)

            pltpu.async_copy(sps[p], shared_tab.at[srows(g)], psems[p])
        for p in range(2):
            pltpu.make_async_copy(
                sps[p], shared_tab.at[srows(0)], psems[p]).wait()
        pltpu.make_async_copy(
            idx_hbm.at[pl.ds(wid * ipw, ipw)], idx_v, xsem).wait()
        plsc.subcore_barrier()

        def start(c, buf, sem):
            for k in range(ci // gi):
                pltpu.async_copy(
                    shared_tab.at[idx_v.at[pl.ds(c * ci + k * gi, gi)]],
                    buf.at[pl.ds(k * gi, gi)], sem)

        def wait(buf, sem):
            pltpu.make_async_copy(
                shared_tab.at[idx_v.at[pl.ds(0, ci)]], buf, sem).wait()

        nh = w2 // LANES

        def load_bf(buf, row, h):
            return plsc.bitcast(buf[row, pl.ds(h * LANES, LANES)], jnp.bfloat16)

        def accum(buf, ostg):
            @pl.loop(0, ca)
            def _(a):
                base = a * MAX_NB

                def rbody(r, accs):
                    return tuple(accs[h] + load_bf(buf, base + r, h)
                                 for h in range(nh))

                accs = tuple(load_bf(buf, base, h) for h in range(nh))
                accs = lax.fori_loop(1, MAX_NB, rbody, accs, unroll=4)
                for h in range(nh):
                    av, bv = plsc.unpack(accs[h],
                                         format=plsc.PackFormat.INTERLEAVED)
                    ostg[a, pl.ds(2 * h * LANES, LANES)] = av
                    ostg[a, pl.ds((2 * h + 1) * LANES, LANES)] = bv

        def owait(ostg, osem):
            pltpu.make_async_copy(ostg, out_hbm.at[pl.ds(0, ca)], osem).wait()

        for b in range(nbuf):
            start(b, bufs[b], sems[b])

        @pl.loop(0, nchunk, step=nbuf)
        def _(c):
            for b in range(nbuf):
                wait(bufs[b], sems[b])

                @pl.when(c + b >= nbuf)
                def _(b=b):
                    owait(ostgs[b], osems[b])

                accum(bufs[b], ostgs[b])
                pltpu.async_copy(
                    ostgs[b],
                    out_hbm.at[pl.ds(wid * apw + (c + b) * ca, ca)], osems[b])

                @pl.when(c + b + nbuf < nchunk)
                def _(b=b):
                    start(c + b + nbuf, bufs[b], sems[b])

        for b in range(nbuf):
            owait(ostgs[b], osems[b])

    cp = pltpu.CompilerParams(use_tc_tiling_on_sc=False)
    if "needs_layout_passes" in pltpu.CompilerParams.__dataclass_fields__:
        cp = dataclasses.replace(cp, needs_layout_passes=False)
    kfn = pl.kernel(
        body,
        out_type=jax.ShapeDtypeStruct((atoms_pad, H), jnp.float32),
        compiler_params=cp,
        mesh=mesh,
        scratch_types=(
            [pltpu.VMEM((ipw,), jnp.int32),
             pltpu.VMEM_SHARED((n_rows, w2), jnp.int32),
             pltpu.VMEM((sr, H), jnp.float32),
             pltpu.VMEM((sr, H), jnp.float32),
             pltpu.VMEM((sr, w2), jnp.int32),
             pltpu.VMEM((sr, w2), jnp.int32),
             pltpu.SemaphoreType.DMA,
             pltpu.SemaphoreType.DMA,
             pltpu.SemaphoreType.DMA,
             pltpu.SemaphoreType.DMA,
             pltpu.SemaphoreType.DMA]
            + [pltpu.VMEM((ci, w2), jnp.int32) for _ in range(nbuf)]
            + [pltpu.VMEM((ca, H), jnp.float32) for _ in range(nbuf)]
            + [pltpu.SemaphoreType.DMA for _ in range(2 * nbuf)]
        ),
    )
    return kfn(table, idx_flat)


def _tc_input(f_atoms, W_i):
    n = f_atoms.shape[0]

    def body(f_ref, w_ref, inp_ref, msg_ref):
        inp = _mm_t(f_ref[...], w_ref[...])
        inp_ref[...] = inp
        msg_ref[...] = jnp.maximum(inp, 0.0)

    return pl.pallas_call(
        body,
        out_shape=(jax.ShapeDtypeStruct((n, H), jnp.float32),
                   jax.ShapeDtypeStruct((n, H), jnp.float32)),
    )(f_atoms, W_i)


def _tc_update(inp, msum_pad, W_h):
    n = inp.shape[0]

    def body(inp_ref, ms_ref, w_ref, out_ref):
        msum = ms_ref[...][:n]
        out_ref[...] = jnp.maximum(
            inp_ref[...] + _mm_t(msum, w_ref[...]), 0.0)

    return pl.pallas_call(
        body,
        out_shape=jax.ShapeDtypeStruct((n, H), jnp.float32),
    )(inp, msum_pad, W_h)


def _tc_head(f_atoms, am, mol_ids2d, e0, e1, W_o, b_o,
             Wf_i, bf_i, Wf1, bf1, Wf2, bf2, n_mols):
    n = f_atoms.shape[0]
    npairs = e0.shape[0]

    def body(f_ref, am_ref, mol_ref, e0_ref, e1_ref, wo_ref, bo_ref,
             wfi_ref, bfi_ref, wf1_ref, bf1_ref, wf2_ref, bf2_ref, out_ref):
        am = am_ref[...][:n]
        wo = wo_ref[...]
        ah = jnp.maximum(
            _mm_t(f_ref[...], wo[:, :H]) + _mm_t(am, wo[:, H:])
            + bo_ref[...], 0.0)
        seg = lax.broadcasted_iota(jnp.int32, (n_mols, n), 0)
        ohm = (seg == mol_ref[...]).astype(jnp.float32)
        sums = jnp.dot(ohm, ah, preferred_element_type=jnp.float32)
        counts = jnp.sum(ohm, axis=1, keepdims=True)
        mv = sums / jnp.maximum(counts, 1.0)

        pid = lax.broadcasted_iota(jnp.int32, (npairs, n_mols), 1)
        oh0 = (pid == e0_ref[...]).astype(jnp.float32)
        oh1 = (pid == e1_ref[...]).astype(jnp.float32)
        v1 = jnp.dot(oh0, mv, preferred_element_type=jnp.float32)
        v2 = jnp.dot(oh1, mv, preferred_element_type=jnp.float32)

        wfi = wfi_ref[...]
        fused = (_mm_t(v1 + v2, wfi[:, :H]) + _mm_t(v1 * v2, wfi[:, H:2 * H])
                 + _mm_t(v1, wfi[:, 2 * H:3 * H]) + _mm_t(v2, wfi[:, 3 * H:])
                 + bfi_ref[...])
        x = jnp.maximum(fused, 0.0)
        x = jnp.maximum(_mm_t(x, wf1_ref[...]) + bf1_ref[...], 0.0)
        logit = jnp.sum(x * wf2_ref[...], axis=1, keepdims=True) + bf2_ref[0, 0]
        out_ref[...] = jax.nn.sigmoid(logit)

    return pl.pallas_call(
        body,
        out_shape=jax.ShapeDtypeStruct((npairs, 1), jnp.float32),
    )(f_atoms, am, mol_ids2d, e0, e1, W_o, b_o,
      Wf_i, bf_i, Wf1, bf1, Wf2, bf2)


def kernel(f_atoms, a_neighbors, mol_ids, batch_edges, W_i, W_h, W_o, b_o,
           Wf_i, bf_i, Wf1, bf1, Wf2, bf2):
    n = f_atoms.shape[0]
    n_mols = 256
    atoms_pad = -(-n // (NW * 8)) * (NW * 8)

    nb = a_neighbors.astype(jnp.int32)
    nb_pad = jnp.pad(nb, ((0, atoms_pad - n), (0, 0)))
    idx_flat = nb_pad.reshape(-1)

    inp, msgp = _tc_input(f_atoms, W_i)
    msump = _gather_sum_sc(msgp, idx_flat, atoms_pad)
    msg2p = _tc_update(inp, msump, W_h)
    am2 = _gather_sum_sc(msg2p, idx_flat, atoms_pad)

    mol_ids2d = mol_ids.astype(jnp.int32).reshape(1, n)
    e0 = batch_edges[0].astype(jnp.int32).reshape(-1, 1)
    e1 = batch_edges[1].astype(jnp.int32).reshape(-1, 1)

    preds = _tc_head(f_atoms, am2, mol_ids2d, e0, e1, W_o,
                     b_o.reshape(1, H), Wf_i, bf_i.reshape(1, -1),
                     Wf1, bf1.reshape(1, -1), Wf2, bf2.reshape(1, -1), n_mols)
    return preds.reshape(-1)
